# Initial kernel scaffold; baseline (speedup 1.0000x reference)
import jax
import jax.numpy as jnp
from jax.experimental import pallas as pl


def kernel(x, adj, W0, W1, W2, a0, a1, a2, W_out, a_out):
    raise NotImplementedError("write your pallas kernel here")


# speedup vs baseline: 1644.1407x; 1644.1407x over previous
import jax
import jax.numpy as jnp
from jax.experimental import pallas as pl

ALPHA = 0.2
N = 1024
NHID = 64


def _leaky(x):
    return jnp.where(x >= 0, x, ALPHA * x)


def _elu(x):
    return jnp.where(x >= 0, x, jnp.exp(x) - 1.0)


def _gat_layer(h, avec, mask, fdim):
    al = avec[:fdim].reshape(fdim, 1)
    ar = avec[fdim:].reshape(fdim, 1)
    f = jnp.dot(h, al, preferred_element_type=jnp.float32)
    g = jnp.dot(h, ar, preferred_element_type=jnp.float32)
    logits = f + g.reshape(1, N)
    S = jnp.exp(-_leaky(logits)) * mask
    rowsum = jnp.sum(S, axis=1, keepdims=True)
    agg = jnp.dot(S, h, preferred_element_type=jnp.float32)
    return agg / rowsum


def _body(x_ref, m_ref, Ws_ref, As_ref, Wo_ref, ao_ref, out_ref):
    x = x_ref[...]
    mask = m_ref[...]
    heads = []
    for k in range(3):
        h = jnp.dot(x, Ws_ref[k], preferred_element_type=jnp.float32)
        heads.append(_elu(_gat_layer(h, As_ref[k], mask, NHID)))
    hcat = jnp.concatenate(heads, axis=1)
    ho = jnp.dot(hcat, Wo_ref[...], preferred_element_type=jnp.float32)
    out = _elu(_gat_layer(ho, ao_ref[...].reshape(-1), mask, ho.shape[1]))
    out_ref[...] = jax.nn.log_softmax(out, axis=1)


def kernel(x, adj, W0, W1, W2, a0, a1, a2, W_out, a_out):
    mask = adj.astype(jnp.float32)
    Ws = jnp.stack([W0, W1, W2])
    As = jnp.stack([a0[0], a1[0], a2[0]])
    nclass = W_out.shape[1]
    return pl.pallas_call(
        _body,
        out_shape=jax.ShapeDtypeStruct((N, nclass), jnp.float32),
    )(x, mask, Ws, As, W_out, a_out)


# speedup vs baseline: 2042.2080x; 1.2421x over previous
import jax
import jax.numpy as jnp
from jax.experimental import pallas as pl

ALPHA = 0.2
N = 1024
NHID = 64
NHEADS = 3


def _leaky(x):
    return jnp.where(x >= 0, x, ALPHA * x)


def _elu(x):
    return jnp.where(x >= 0, x, jnp.exp(x) - 1.0)


def _gat_layer(h, al, ar, mask):
    f = jnp.dot(h, al, preferred_element_type=jnp.float32)
    g = jnp.dot(h, ar, preferred_element_type=jnp.float32)
    logits = f + g.reshape(1, N)
    S = jnp.exp(-_leaky(logits)) * mask
    rowsum = jnp.sum(S, axis=1, keepdims=True)
    agg = jnp.dot(S, h, preferred_element_type=jnp.float32)
    return agg / rowsum


def _body(x_ref, m_ref, Wall_ref, As_ref, Wo_ref, ao_ref, out_ref):
    x = x_ref[...]
    mask = m_ref[...].astype(jnp.float32)
    H = jnp.dot(x, Wall_ref[...], preferred_element_type=jnp.float32)
    heads = []
    for k in range(NHEADS):
        h = H[:, k * NHID:(k + 1) * NHID]
        al = As_ref[k, :NHID].reshape(NHID, 1)
        ar = As_ref[k, NHID:].reshape(NHID, 1)
        heads.append(_elu(_gat_layer(h, al, ar, mask)))
    hcat = jnp.concatenate(heads, axis=1)
    ho = jnp.dot(hcat, Wo_ref[...], preferred_element_type=jnp.float32)
    nc = ho.shape[1]
    ao = ao_ref[...].reshape(-1)
    out = _elu(_gat_layer(ho, ao[:nc].reshape(nc, 1), ao[nc:].reshape(nc, 1), mask))
    out_ref[...] = jax.nn.log_softmax(out, axis=1)


def kernel(x, adj, W0, W1, W2, a0, a1, a2, W_out, a_out):
    mask_i = adj.astype(jnp.int32)
    W_all = jnp.concatenate([W0, W1, W2], axis=1)
    As = jnp.stack([a0[0], a1[0], a2[0]])
    nclass = W_out.shape[1]
    return pl.pallas_call(
        _body,
        out_shape=jax.ShapeDtypeStruct((N, nclass), jnp.float32),
    )(x, mask_i, W_all, As, W_out, a_out)


# speedup vs baseline: 2080.0521x; 1.0185x over previous
import jax
import jax.numpy as jnp
from jax.experimental import pallas as pl

ALPHA = 0.2
N = 1024
NHID = 64
NHEADS = 3
LOG2E = 1.4426950408889634


def _elu(x):
    return jnp.where(x >= 0, x, jnp.exp(x) - 1.0)


def _gat_layer(h, al, ar, maskf, ones_col):
    f = jnp.dot(h, al, preferred_element_type=jnp.float32)
    g = jnp.dot(h, ar, preferred_element_type=jnp.float32)
    z = f + g.reshape(1, N)
    slope = jnp.where(z >= 0, -LOG2E, -ALPHA * LOG2E)
    S = jnp.exp2(z * slope) * maskf
    hp = jnp.concatenate([h, ones_col], axis=1)
    agg = jnp.dot(S, hp, preferred_element_type=jnp.float32)
    rinv = 1.0 / agg[:, h.shape[1]:h.shape[1] + 1]
    return agg[:, :h.shape[1]] * rinv


def _body(x_ref, m_ref, W0_ref, W1_ref, W2_ref, a0_ref, a1_ref, a2_ref,
          Wo_ref, ao_ref, out_ref):
    x = x_ref[...]
    maskf = m_ref[...].astype(jnp.float32)
    ones_col = jnp.ones((N, 1), jnp.float32)
    W_all = jnp.concatenate([W0_ref[...], W1_ref[...], W2_ref[...]], axis=1)
    H = jnp.dot(x, W_all, preferred_element_type=jnp.float32)
    a_refs = (a0_ref, a1_ref, a2_ref)
    heads = []
    for k in range(NHEADS):
        h = H[:, k * NHID:(k + 1) * NHID]
        ak = a_refs[k][...].reshape(-1)
        al = ak[:NHID].reshape(NHID, 1)
        ar = ak[NHID:].reshape(NHID, 1)
        heads.append(_elu(_gat_layer(h, al, ar, maskf, ones_col)))
    hcat = jnp.concatenate(heads, axis=1)
    ho = jnp.dot(hcat, Wo_ref[...], preferred_element_type=jnp.float32)
    nc = ho.shape[1]
    ao = ao_ref[...].reshape(-1)
    out = _elu(_gat_layer(ho, ao[:nc].reshape(nc, 1),
                          ao[nc:].reshape(nc, 1), maskf, ones_col))
    out_ref[...] = jax.nn.log_softmax(out, axis=1)


def kernel(x, adj, W0, W1, W2, a0, a1, a2, W_out, a_out):
    nclass = W_out.shape[1]
    return pl.pallas_call(
        _body,
        out_shape=jax.ShapeDtypeStruct((N, nclass), jnp.float32),
    )(x, adj.astype(jnp.int32), W0, W1, W2, a0, a1, a2, W_out, a_out)


# speedup vs baseline: 3716.1494x; 1.7866x over previous
import jax
import jax.numpy as jnp
from jax.experimental import pallas as pl
from jax.experimental.pallas import tpu as pltpu

ALPHA = 0.2
N = 1024
NFEAT = 256
NHID = 64
NHEADS = 3
NCLASS = 40
LOG2E = 1.4426950408889634

_DOT_T = (((1,), (1,)), ((), ()))


def _elu(x):
    return jnp.where(x >= 0, x, jnp.exp(x) - 1.0)


def _gat_layer(h, al, ar, maskf, ones_col):
    f = jnp.dot(h, al, preferred_element_type=jnp.float32)
    g = jnp.dot(h, ar, preferred_element_type=jnp.float32)
    z = f + g.reshape(1, N)
    slope = jnp.where(z >= 0, -LOG2E, -ALPHA * LOG2E)
    S = jnp.exp2(z * slope) * maskf
    hp = jnp.concatenate([h, ones_col], axis=1)
    agg = jnp.dot(S, hp, preferred_element_type=jnp.float32)
    rinv = 1.0 / agg[:, h.shape[1]:h.shape[1] + 1]
    return agg[:, :h.shape[1]] * rinv


def _body(x_hbm, m_hbm, W0_h, W1_h, W2_h, a0_h, a1_h, a2_h, Wo_h, ao_h,
          out_ref, xv, mv, Wv, av, Wov, aov, sems):
    cps = [
        pltpu.make_async_copy(x_hbm, xv, sems.at[0]),
        pltpu.make_async_copy(W0_h, Wv.at[0], sems.at[1]),
        pltpu.make_async_copy(W1_h, Wv.at[1], sems.at[2]),
        pltpu.make_async_copy(W2_h, Wv.at[2], sems.at[3]),
        pltpu.make_async_copy(a0_h, av.at[0:1, :], sems.at[4]),
        pltpu.make_async_copy(a1_h, av.at[1:2, :], sems.at[5]),
        pltpu.make_async_copy(a2_h, av.at[2:3, :], sems.at[6]),
        pltpu.make_async_copy(Wo_h, Wov, sems.at[7]),
        pltpu.make_async_copy(ao_h, aov.at[0:1, :], sems.at[8]),
        pltpu.make_async_copy(m_hbm, mv, sems.at[9]),
    ]
    for c in cps:
        c.start()
    for c in cps[:-1]:
        c.wait()

    x = xv[...]
    ones_col = jnp.ones((N, 1), jnp.float32)
    hs = [jax.lax.dot_general(x, Wv[k], _DOT_T,
                              preferred_element_type=jnp.float32)
          for k in range(NHEADS)]
    cps[-1].wait()
    maskf = mv[...].astype(jnp.float32)
    heads = []
    for k in range(NHEADS):
        h = hs[k]
        ak = av[k, :]
        al = ak[:NHID].reshape(NHID, 1)
        ar = ak[NHID:].reshape(NHID, 1)
        heads.append(_elu(_gat_layer(h, al, ar, maskf, ones_col)))
    hcat = jnp.concatenate(heads, axis=1)
    ho = jax.lax.dot_general(hcat, Wov[...], _DOT_T,
                             preferred_element_type=jnp.float32)
    ao = aov[0, :]
    out = _elu(_gat_layer(ho, ao[:NCLASS].reshape(NCLASS, 1),
                          ao[NCLASS:2 * NCLASS].reshape(NCLASS, 1),
                          maskf, ones_col))
    out_ref[...] = jax.nn.log_softmax(out, axis=1).T


def kernel(x, adj, W0, W1, W2, a0, a1, a2, W_out, a_out):
    res = pl.pallas_call(
        _body,
        in_specs=[pl.BlockSpec(memory_space=pl.ANY)] * 10,
        out_shape=jax.ShapeDtypeStruct((NCLASS, N), jnp.float32),
        scratch_shapes=[
            pltpu.VMEM((N, NFEAT), jnp.float32),
            pltpu.VMEM((N, N), jnp.int32),
            pltpu.VMEM((NHEADS, NHID, NFEAT), jnp.float32),
            pltpu.VMEM((NHEADS, 2 * NHID), jnp.float32),
            pltpu.VMEM((NCLASS, NHID * NHEADS), jnp.float32),
            pltpu.VMEM((1, 2 * NCLASS), jnp.float32),
            pltpu.SemaphoreType.DMA((10,)),
        ],
    )(*[pltpu.with_memory_space_constraint(v, pltpu.MemorySpace.HBM)
        for v in (x, adj.astype(jnp.int32), W0.T, W1.T, W2.T, a0, a1, a2,
                  W_out.T, a_out)])
    return res.T
